# baseline (speedup 1.0000x reference)
import functools

import jax
import jax.numpy as jnp
from jax import lax
from jax.experimental import pallas as pl
from jax.experimental.pallas import tpu as pltpu
from jax.experimental.pallas import tpu_sc as plsc

N = 10000
E = 320000
H = 128
NG = 25
NODE_SZ = 400

NC = 2
NS = 16
NW = NC * NS

NP = 10240
ROWS_PER_TILE = NP // NS
CHUNK = 96
EW_PER = 10176
NCHUNK = EW_PER // CHUNK
EP = EW_PER * NW

BN = 1024


def _leaky(v):
    return jnp.where(v >= 0, v, 0.01 * v)


_MESH = functools.partial(
    plsc.VectorSubcoreMesh, core_axis_name="c", subcore_axis_name="s",
    num_cores=NC, num_subcores=NS)


def _deg_body(col_hbm, ew_hbm, deg_out, ewc_out,
              col_v0, col_v1, ew_all, zb_v, deg_sh, csem0, csem1, psem):
    cid = lax.axis_index("c")
    sid = lax.axis_index("s")
    wid = sid * NC + cid
    ebase = wid * EW_PER
    col_v = (col_v0, col_v1)
    csem = (csem0, csem1)

    z16 = jnp.zeros((16,), jnp.float32)
    for i in range(ROWS_PER_TILE // 16):
        zb_v[pl.ds(i * 16, 16)] = z16
    pltpu.sync_copy(zb_v, deg_sh.at[pl.ds(sid * ROWS_PER_TILE, ROWS_PER_TILE)])

    pltpu.sync_copy(ew_hbm.at[pl.ds(ebase, EW_PER)], ew_all)

    @plsc.parallel_loop(0, EW_PER // 16, unroll=4)
    def _(i):
        sl = pl.ds(i * 16, 16)
        ew_all[sl] = jnp.maximum(ew_all[sl], 0.0)

    pltpu.sync_copy(ew_all, ewc_out.at[pl.ds(ebase, EW_PER)])
    plsc.subcore_barrier()

    def col_load(k, b):
        return pltpu.async_copy(
            col_hbm.at[pl.ds(ebase + k * CHUNK, CHUNK)], col_v[b], csem[b])

    col_load(0, 0)

    def pair(kk, carry):
        for b in range(2):
            k = kk * 2 + b
            pltpu.make_async_copy(
                col_hbm.at[pl.ds(ebase + k * CHUNK, CHUNK)],
                col_v[b], csem[b]).wait()
            col_load(k + 1, 1 - b)
            pltpu.sync_copy(ew_all.at[pl.ds(k * CHUNK, CHUNK)],
                            deg_sh.at[col_v[b]], add=True)
        return carry

    lax.fori_loop(0, (NCHUNK - 2) // 2, pair, 0)
    for k in (NCHUNK - 2, NCHUNK - 1):
        b = k % 2
        pltpu.make_async_copy(
            col_hbm.at[pl.ds(ebase + k * CHUNK, CHUNK)],
            col_v[b], csem[b]).wait()
        if k == NCHUNK - 2:
            col_load(k + 1, 1 - b)
        pltpu.sync_copy(ew_all.at[pl.ds(k * CHUNK, CHUNK)],
                        deg_sh.at[col_v[b]], add=True)

    plsc.subcore_barrier()
    sl = pl.ds(sid * ROWS_PER_TILE, ROWS_PER_TILE)
    pltpu.sync_copy(deg_sh.at[sl], deg_out.at[cid, sl])


def _deg_call(colp, ewp):
    k = pl.kernel(
        _deg_body,
        out_type=(jax.ShapeDtypeStruct((NC, NP), jnp.float32),
                  jax.ShapeDtypeStruct((EP,), jnp.float32)),
        mesh=_MESH(),
        scratch_types=[
            pltpu.VMEM((CHUNK,), jnp.int32),
            pltpu.VMEM((CHUNK,), jnp.int32),
            pltpu.VMEM((EW_PER,), jnp.float32),
            pltpu.VMEM((ROWS_PER_TILE,), jnp.float32),
            pltpu.VMEM_SHARED((NP,), jnp.float32),
            pltpu.SemaphoreType.DMA,
            pltpu.SemaphoreType.DMA,
            pltpu.SemaphoreType.DMA,
        ],
    )
    return k(colp, ewp)


NBUF = 3


def _make_spmm_body():
    def body(hs_hbm, row_hbm, col_hbm, w_hbm, out_hbm,
             rv0, rv1, rv2, cv0, cv1, cv2, wv0, wv1, wv2,
             rb0, rb1, rb2, acc_sh,
             i0, i1, i2, g0, g1, g2, s0, s1, s2):
        cid = lax.axis_index("c")
        sid = lax.axis_index("s")
        wid = sid * NC + cid
        ebase = wid * EW_PER
        row_v = (rv0, rv1, rv2)
        col_v = (cv0, cv1, cv2)
        w_v = (wv0, wv1, wv2)
        rows = (rb0, rb1, rb2)
        isem = (i0, i1, i2)
        gsem = (g0, g1, g2)
        ssem = (s0, s1, s2)

        z16 = jnp.zeros((16,), jnp.float32)

        def zrow(i, carry):
            for j in range(H // 16):
                rb0[i, pl.ds(j * 16, 16)] = z16
            return carry

        lax.fori_loop(0, CHUNK, zrow, 0)
        zbase = sid * ROWS_PER_TILE
        nfull = ROWS_PER_TILE // CHUNK
        for kk in range(nfull):
            pltpu.sync_copy(rb0, acc_sh.at[pl.ds(zbase + kk * CHUNK, CHUNK)])
        rem = ROWS_PER_TILE - nfull * CHUNK
        if rem:
            pltpu.sync_copy(rb0.at[pl.ds(0, rem)],
                            acc_sh.at[pl.ds(zbase + nfull * CHUNK, rem)])
        plsc.subcore_barrier()

        def issue_idx(k, b):
            base = ebase + k * CHUNK
            pltpu.async_copy(row_hbm.at[pl.ds(base, CHUNK)], row_v[b], isem[b])
            pltpu.async_copy(col_hbm.at[pl.ds(base, CHUNK)], col_v[b], isem[b])
            pltpu.async_copy(w_hbm.at[pl.ds(base, CHUNK)], w_v[b], isem[b])

        def wait_idx(k, b):
            base = ebase + k * CHUNK
            pltpu.make_async_copy(row_hbm.at[pl.ds(base, CHUNK)],
                                  row_v[b], isem[b]).wait()
            pltpu.make_async_copy(col_hbm.at[pl.ds(base, CHUNK)],
                                  col_v[b], isem[b]).wait()
            pltpu.make_async_copy(w_hbm.at[pl.ds(base, CHUNK)],
                                  w_v[b], isem[b]).wait()

        def gstart(b):
            pltpu.async_copy(hs_hbm.at[row_v[b]], rows[b], gsem[b])

        def gwait(b):
            pltpu.make_async_copy(hs_hbm.at[row_v[b]], rows[b],
                                  gsem[b]).wait()

        def wait_scatter(b):
            pltpu.make_async_copy(rows[b], acc_sh.at[col_v[b]],
                                  ssem[b]).wait()

        def process(k, b):
            gwait(b)
            rv = rows[b]
            wv = w_v[b]

            @plsc.parallel_loop(0, CHUNK // 16, unroll=3)
            def _(g):
                sv = wv[pl.ds(g * 16, 16)]
                for l in range(16):
                    bc = jnp.full((16,), sv[l], dtype=jnp.float32)
                    e = g * 16 + l
                    for j in range(H // 16):
                        sl = pl.ds(j * 16, 16)
                        rv[e, sl] = rv[e, sl] * bc

            pltpu.async_copy(rv, acc_sh.at[col_v[b]], ssem[b], add=True)

        issue_idx(0, 0)
        issue_idx(1, 1)
        wait_idx(0, 0)
        gstart(0)
        wait_idx(1, 1)
        gstart(1)
        process(0, 0)
        issue_idx(2, 2)
        wait_idx(2, 2)
        gstart(2)
        process(1, 1)
        wait_scatter(0)
        issue_idx(3, 0)

        def steady(kk, carry):
            for j in range(3):
                k = 2 + kk * 3 + j
                b = (2 + j) % 3
                bn = j % 3
                b2 = (1 + j) % 3
                wait_idx(k + 1, bn)
                gstart(bn)
                process(k, b)
                wait_scatter(b2)
                issue_idx(k + 2, b2)
            return carry

        lax.fori_loop(0, (NCHUNK - 4) // 3, steady, 0)
        wait_idx(NCHUNK - 1, (NCHUNK - 1) % 3)
        gstart((NCHUNK - 1) % 3)
        process(NCHUNK - 2, (NCHUNK - 2) % 3)
        process(NCHUNK - 1, (NCHUNK - 1) % 3)
        for b in range(NBUF):
            wait_scatter(b)

        plsc.subcore_barrier()
        for kk in range(nfull):
            sl = pl.ds(zbase + kk * CHUNK, CHUNK)
            pltpu.sync_copy(acc_sh.at[sl], out_hbm.at[cid, sl])
        if rem:
            sl = pl.ds(zbase + nfull * CHUNK, rem)
            pltpu.sync_copy(acc_sh.at[sl], out_hbm.at[cid, sl])

    return body


def _spmm_call(hs, rowp, colp, ewc):
    k = pl.kernel(
        _make_spmm_body(),
        out_type=jax.ShapeDtypeStruct((NC, NP, H), jnp.float32),
        mesh=_MESH(),
        scratch_types=(
            [pltpu.VMEM((CHUNK,), jnp.int32) for _ in range(NBUF)]
            + [pltpu.VMEM((CHUNK,), jnp.int32) for _ in range(NBUF)]
            + [pltpu.VMEM((CHUNK,), jnp.float32) for _ in range(NBUF)]
            + [pltpu.VMEM((CHUNK, H), jnp.float32) for _ in range(NBUF)]
            + [pltpu.VMEM_SHARED((NP, H), jnp.float32)]
            + [pltpu.SemaphoreType.DMA for _ in range(3 * NBUF)]
        ),
    )
    return k(hs, rowp, colp, ewc)


def _mm_scale_kernel(x_ref, w_ref, deg_ref, o_ref):
    h = jnp.dot(x_ref[...], w_ref[...], preferred_element_type=jnp.float32)
    dinv = lax.rsqrt(deg_ref[0] + deg_ref[1] + 1.0)
    o_ref[...] = h * dinv


def _mid_kernel(s_ref, hs_ref, deg_ref, w_ref, b_ref, o_ref):
    dinv = lax.rsqrt(deg_ref[0] + deg_ref[1] + 1.0)
    s = s_ref[0] + s_ref[1]
    h1 = _leaky(dinv * (s + hs_ref[...]) + b_ref[...])
    o_ref[...] = dinv * jnp.dot(h1, w_ref[...],
                                preferred_element_type=jnp.float32)


def _fin_kernel(s_ref, hs_ref, deg_ref, b_ref, o_ref):
    dinv = lax.rsqrt(deg_ref[0] + deg_ref[1] + 1.0)
    s = s_ref[0] + s_ref[1]
    o_ref[...] = dinv * (s + hs_ref[...]) + b_ref[...]


def _readout_kernel(h2_ref, wlc_ref, blc_ref, wf1_ref, bf1_ref,
                    wf2_ref, bf2_ref, wf3_ref, bf3_ref, o_ref, acc_ref):
    k = pl.program_id(0)

    @pl.when(k == 0)
    def _():
        acc_ref[...] = jnp.zeros_like(acc_ref)

    acc_ref[...] += jnp.dot(h2_ref[...], wlc_ref[...],
                            preferred_element_type=jnp.float32)

    @pl.when(k == pl.num_programs(0) - 1)
    def _():
        g = acc_ref[...] + blc_ref[...]
        g = _leaky(jnp.dot(g, wf1_ref[...],
                           preferred_element_type=jnp.float32) + bf1_ref[...])
        g = _leaky(jnp.dot(g, wf2_ref[...],
                           preferred_element_type=jnp.float32) + bf2_ref[...])
        o_ref[...] = jnp.dot(g, wf3_ref[...],
                             preferred_element_type=jnp.float32) + bf3_ref[...]


def _mm_scale(xp, W1, deg3):
    grid = NP // BN
    return pl.pallas_call(
        _mm_scale_kernel,
        grid=(grid,),
        in_specs=[
            pl.BlockSpec((BN, H), lambda i: (i, 0)),
            pl.BlockSpec((H, H), lambda i: (0, 0)),
            pl.BlockSpec((NC, BN, 1), lambda i: (0, i, 0)),
        ],
        out_specs=pl.BlockSpec((BN, H), lambda i: (i, 0)),
        out_shape=jax.ShapeDtypeStruct((NP, H), jnp.float32),
    )(xp, W1, deg3)


def _mid(S1, hs1, deg3, W2, b1):
    grid = NP // BN
    return pl.pallas_call(
        _mid_kernel,
        grid=(grid,),
        in_specs=[
            pl.BlockSpec((NC, BN, H), lambda i: (0, i, 0)),
            pl.BlockSpec((BN, H), lambda i: (i, 0)),
            pl.BlockSpec((NC, BN, 1), lambda i: (0, i, 0)),
            pl.BlockSpec((H, H), lambda i: (0, 0)),
            pl.BlockSpec((1, H), lambda i: (0, 0)),
        ],
        out_specs=pl.BlockSpec((BN, H), lambda i: (i, 0)),
        out_shape=jax.ShapeDtypeStruct((NP, H), jnp.float32),
    )(S1, hs1, deg3, W2, b1)


def _fin(S2, hs2, deg3, b2):
    grid = NP // BN
    return pl.pallas_call(
        _fin_kernel,
        grid=(grid,),
        in_specs=[
            pl.BlockSpec((NC, BN, H), lambda i: (0, i, 0)),
            pl.BlockSpec((BN, H), lambda i: (i, 0)),
            pl.BlockSpec((NC, BN, 1), lambda i: (0, i, 0)),
            pl.BlockSpec((1, H), lambda i: (0, 0)),
        ],
        out_specs=pl.BlockSpec((BN, H), lambda i: (i, 0)),
        out_shape=jax.ShapeDtypeStruct((NP, H), jnp.float32),
    )(S2, hs2, deg3, b2)


def _readout(h2r, Wlc, blc, Wf1, bf1, Wf2, bf2, Wf3, bf3):
    BK = 2048
    grid = (NODE_SZ * H) // BK
    return pl.pallas_call(
        _readout_kernel,
        grid=(grid,),
        in_specs=[
            pl.BlockSpec((NG, BK), lambda k: (0, k)),
            pl.BlockSpec((BK, H), lambda k: (k, 0)),
            pl.BlockSpec((1, H), lambda k: (0, 0)),
            pl.BlockSpec((H, H // 2), lambda k: (0, 0)),
            pl.BlockSpec((1, H // 2), lambda k: (0, 0)),
            pl.BlockSpec((H // 2, H // 4), lambda k: (0, 0)),
            pl.BlockSpec((1, H // 4), lambda k: (0, 0)),
            pl.BlockSpec((H // 4, 1), lambda k: (0, 0)),
            pl.BlockSpec((1, 1), lambda k: (0, 0)),
        ],
        out_specs=pl.BlockSpec((NG, 1), lambda k: (0, 0)),
        out_shape=jax.ShapeDtypeStruct((NG, 1), jnp.float32),
        scratch_shapes=[pltpu.VMEM((NG, H), jnp.float32)],
    )(h2r, Wlc, blc, Wf1, bf1, Wf2, bf2, Wf3, bf3)


def kernel(x, edge_index, edge_weight, batch,
           W1, b1, W2, b2, Wlc, blc, Wf1, bf1, Wf2, bf2, Wf3, bf3):
    row, col = edge_index[0], edge_index[1]

    pad_e = EP - E
    pad_idx = (jnp.arange(pad_e, dtype=jnp.int32) * 2503 + 17) % N
    rowp = jnp.concatenate([row, pad_idx])
    colp = jnp.concatenate([col, pad_idx])
    ewp = jnp.concatenate([edge_weight, jnp.zeros((pad_e,), jnp.float32)])
    xp = jnp.pad(x, ((0, NP - N), (0, 0)))

    deg2, ewc = _deg_call(colp, ewp)
    deg3 = deg2.reshape(NC, NP, 1)

    hs1 = _mm_scale(xp, W1, deg3)
    S1 = _spmm_call(hs1, rowp, colp, ewc)
    hs2 = _mid(S1, hs1, deg3, W2, b1.reshape(1, H))
    S2 = _spmm_call(hs2, rowp, colp, ewc)
    h2 = _fin(S2, hs2, deg3, b2.reshape(1, H))

    h2r = h2[:N].reshape(NG, NODE_SZ * H)
    return _readout(h2r, Wlc, blc.reshape(1, H),
                    Wf1, bf1.reshape(1, H // 2),
                    Wf2, bf2.reshape(1, H // 4),
                    Wf3, bf3.reshape(1, 1))


# speedup vs baseline: 1.0544x; 1.0544x over previous
import functools

import jax
import jax.numpy as jnp
from jax import lax
from jax.experimental import pallas as pl
from jax.experimental.pallas import tpu as pltpu
from jax.experimental.pallas import tpu_sc as plsc

N = 10000
E = 320000
H = 128
NG = 25
NODE_SZ = 400

NC = 2
NS = 16
NW = NC * NS

NP = 10240
ROWS_PER_TILE = NP // NS
CHUNK = 96
EW_PER = 10176
NCHUNK = EW_PER // CHUNK
EP = EW_PER * NW

BN = 1024


def _leaky(v):
    return jnp.where(v >= 0, v, 0.01 * v)


_MESH = functools.partial(
    plsc.VectorSubcoreMesh, core_axis_name="c", subcore_axis_name="s",
    num_cores=NC, num_subcores=NS)


def _deg_body(col_hbm, ew_hbm, deg_out, ewc_out,
              col_v0, col_v1, col_v2, ew_all, zb_v, deg_sh,
              c0, c1, c2, d0, d1, d2):
    cid = lax.axis_index("c")
    sid = lax.axis_index("s")
    wid = sid * NC + cid
    ebase = wid * EW_PER
    col_v = (col_v0, col_v1, col_v2)
    csem = (c0, c1, c2)
    dsem = (d0, d1, d2)

    z16 = jnp.zeros((16,), jnp.float32)
    for i in range(ROWS_PER_TILE // 16):
        zb_v[pl.ds(i * 16, 16)] = z16
    pltpu.sync_copy(zb_v, deg_sh.at[pl.ds(sid * ROWS_PER_TILE, ROWS_PER_TILE)])

    pltpu.sync_copy(ew_hbm.at[pl.ds(ebase, EW_PER)], ew_all)

    @plsc.parallel_loop(0, EW_PER // 16, unroll=4)
    def _(i):
        sl = pl.ds(i * 16, 16)
        ew_all[sl] = jnp.maximum(ew_all[sl], 0.0)

    pltpu.sync_copy(ew_all, ewc_out.at[pl.ds(ebase, EW_PER)])
    plsc.subcore_barrier()

    def col_load(k, b):
        pltpu.async_copy(
            col_hbm.at[pl.ds(ebase + k * CHUNK, CHUNK)], col_v[b], csem[b])

    def col_wait(k, b):
        pltpu.make_async_copy(
            col_hbm.at[pl.ds(ebase + k * CHUNK, CHUNK)],
            col_v[b], csem[b]).wait()

    def scat(k, b):
        pltpu.async_copy(ew_all.at[pl.ds(k * CHUNK, CHUNK)],
                         deg_sh.at[col_v[b]], dsem[b], add=True)

    def scat_wait(k, b):
        pltpu.make_async_copy(ew_all.at[pl.ds(k * CHUNK, CHUNK)],
                              deg_sh.at[col_v[b]], dsem[b]).wait()

    col_load(0, 0)
    col_load(1, 1)
    col_wait(0, 0)
    scat(0, 0)
    col_load(2, 2)
    col_wait(1, 1)
    scat(1, 1)
    scat_wait(0, 0)
    col_load(3, 0)

    def steady(kk, carry):
        for j in range(3):
            k = 2 + kk * 3 + j
            b = (2 + j) % 3
            b2 = (1 + j) % 3
            col_wait(k, b)
            scat(k, b)
            scat_wait(k - 1, b2)
            col_load(k + 2, b2)
        return carry

    lax.fori_loop(0, (NCHUNK - 4) // 3, steady, 0)
    for k in (NCHUNK - 2, NCHUNK - 1):
        b = k % 3
        col_wait(k, b)
        scat(k, b)
    for j, b in ((NCHUNK - 3, (NCHUNK - 3) % 3), (NCHUNK - 2, (NCHUNK - 2) % 3),
                 (NCHUNK - 1, (NCHUNK - 1) % 3)):
        scat_wait(j, b)

    plsc.subcore_barrier()
    sl = pl.ds(sid * ROWS_PER_TILE, ROWS_PER_TILE)
    pltpu.sync_copy(deg_sh.at[sl], deg_out.at[cid, sl])


def _deg_call(colp, ewp):
    k = pl.kernel(
        _deg_body,
        out_type=(jax.ShapeDtypeStruct((NC, NP), jnp.float32),
                  jax.ShapeDtypeStruct((EP,), jnp.float32)),
        mesh=_MESH(),
        scratch_types=(
            [pltpu.VMEM((CHUNK,), jnp.int32) for _ in range(3)]
            + [pltpu.VMEM((EW_PER,), jnp.float32),
               pltpu.VMEM((ROWS_PER_TILE,), jnp.float32),
               pltpu.VMEM_SHARED((NP,), jnp.float32)]
            + [pltpu.SemaphoreType.DMA for _ in range(6)]
        ),
    )
    return k(colp, ewp)


NBUF = 3


def _make_spmm_body():
    def body(hs_hbm, row_hbm, col_hbm, w_hbm, out_hbm,
             rv0, rv1, rv2, cv0, cv1, cv2, wv0, wv1, wv2,
             rb0, rb1, rb2, acc_sh,
             i0, i1, i2, g0, g1, g2, s0, s1, s2):
        cid = lax.axis_index("c")
        sid = lax.axis_index("s")
        wid = sid * NC + cid
        ebase = wid * EW_PER
        row_v = (rv0, rv1, rv2)
        col_v = (cv0, cv1, cv2)
        w_v = (wv0, wv1, wv2)
        rows = (rb0, rb1, rb2)
        isem = (i0, i1, i2)
        gsem = (g0, g1, g2)
        ssem = (s0, s1, s2)

        z16 = jnp.zeros((16,), jnp.float32)

        def zrow(i, carry):
            for j in range(H // 16):
                rb0[i, pl.ds(j * 16, 16)] = z16
            return carry

        lax.fori_loop(0, CHUNK, zrow, 0)
        zbase = sid * ROWS_PER_TILE
        nfull = ROWS_PER_TILE // CHUNK
        for kk in range(nfull):
            pltpu.sync_copy(rb0, acc_sh.at[pl.ds(zbase + kk * CHUNK, CHUNK)])
        rem = ROWS_PER_TILE - nfull * CHUNK
        if rem:
            pltpu.sync_copy(rb0.at[pl.ds(0, rem)],
                            acc_sh.at[pl.ds(zbase + nfull * CHUNK, rem)])
        plsc.subcore_barrier()

        def issue_idx(k, b):
            base = ebase + k * CHUNK
            pltpu.async_copy(row_hbm.at[pl.ds(base, CHUNK)], row_v[b], isem[b])
            pltpu.async_copy(col_hbm.at[pl.ds(base, CHUNK)], col_v[b], isem[b])
            pltpu.async_copy(w_hbm.at[pl.ds(base, CHUNK)], w_v[b], isem[b])

        def wait_idx(k, b):
            base = ebase + k * CHUNK
            pltpu.make_async_copy(row_hbm.at[pl.ds(base, CHUNK)],
                                  row_v[b], isem[b]).wait()
            pltpu.make_async_copy(col_hbm.at[pl.ds(base, CHUNK)],
                                  col_v[b], isem[b]).wait()
            pltpu.make_async_copy(w_hbm.at[pl.ds(base, CHUNK)],
                                  w_v[b], isem[b]).wait()

        def gstart(b):
            pltpu.async_copy(hs_hbm.at[row_v[b]], rows[b], gsem[b])

        def gwait(b):
            pltpu.make_async_copy(hs_hbm.at[row_v[b]], rows[b],
                                  gsem[b]).wait()

        def wait_scatter(b):
            pltpu.make_async_copy(rows[b], acc_sh.at[col_v[b]],
                                  ssem[b]).wait()

        def process(k, b):
            gwait(b)
            rv = rows[b]
            wv = w_v[b]

            @plsc.parallel_loop(0, CHUNK // 16, unroll=2)
            def _(g):
                sv = wv[pl.ds(g * 16, 16)]
                for l in range(16):
                    bc = jnp.full((16,), sv[l], dtype=jnp.float32)
                    e = g * 16 + l
                    for j in range(H // 16):
                        sl = pl.ds(j * 16, 16)
                        rv[e, sl] = rv[e, sl] * bc

            pltpu.async_copy(rv, acc_sh.at[col_v[b]], ssem[b], add=True)

        issue_idx(0, 0)
        issue_idx(1, 1)
        wait_idx(0, 0)
        gstart(0)
        wait_idx(1, 1)
        gstart(1)
        process(0, 0)
        issue_idx(2, 2)
        wait_idx(2, 2)
        gstart(2)
        process(1, 1)
        wait_scatter(0)
        issue_idx(3, 0)

        def steady(kk, carry):
            for j in range(3):
                k = 2 + kk * 3 + j
                b = (2 + j) % 3
                bn = j % 3
                b2 = (1 + j) % 3
                wait_idx(k + 1, bn)
                gstart(bn)
                process(k, b)
                wait_scatter(b2)
                issue_idx(k + 2, b2)
            return carry

        lax.fori_loop(0, (NCHUNK - 4) // 3, steady, 0)
        wait_idx(NCHUNK - 1, (NCHUNK - 1) % 3)
        gstart((NCHUNK - 1) % 3)
        process(NCHUNK - 2, (NCHUNK - 2) % 3)
        process(NCHUNK - 1, (NCHUNK - 1) % 3)
        for b in range(NBUF):
            wait_scatter(b)

        plsc.subcore_barrier()
        for kk in range(nfull):
            sl = pl.ds(zbase + kk * CHUNK, CHUNK)
            pltpu.sync_copy(acc_sh.at[sl], out_hbm.at[cid, sl])
        if rem:
            sl = pl.ds(zbase + nfull * CHUNK, rem)
            pltpu.sync_copy(acc_sh.at[sl], out_hbm.at[cid, sl])

    return body


def _spmm_call(hs, rowp, colp, ewc):
    k = pl.kernel(
        _make_spmm_body(),
        out_type=jax.ShapeDtypeStruct((NC, NP, H), jnp.float32),
        mesh=_MESH(),
        scratch_types=(
            [pltpu.VMEM((CHUNK,), jnp.int32) for _ in range(NBUF)]
            + [pltpu.VMEM((CHUNK,), jnp.int32) for _ in range(NBUF)]
            + [pltpu.VMEM((CHUNK,), jnp.float32) for _ in range(NBUF)]
            + [pltpu.VMEM((CHUNK, H), jnp.float32) for _ in range(NBUF)]
            + [pltpu.VMEM_SHARED((NP, H), jnp.float32)]
            + [pltpu.SemaphoreType.DMA for _ in range(3 * NBUF)]
        ),
    )
    return k(hs, rowp, colp, ewc)


def _mm_scale_kernel(x_ref, w_ref, deg_ref, o_ref):
    h = jnp.dot(x_ref[...], w_ref[...], preferred_element_type=jnp.float32)
    dinv = lax.rsqrt(deg_ref[0] + deg_ref[1] + 1.0)
    o_ref[...] = h * dinv


def _mid_kernel(s_ref, hs_ref, deg_ref, w_ref, b_ref, o_ref):
    dinv = lax.rsqrt(deg_ref[0] + deg_ref[1] + 1.0)
    s = s_ref[0] + s_ref[1]
    h1 = _leaky(dinv * (s + hs_ref[...]) + b_ref[...])
    o_ref[...] = dinv * jnp.dot(h1, w_ref[...],
                                preferred_element_type=jnp.float32)


def _fin_kernel(s_ref, hs_ref, deg_ref, b_ref, o_ref):
    dinv = lax.rsqrt(deg_ref[0] + deg_ref[1] + 1.0)
    s = s_ref[0] + s_ref[1]
    o_ref[...] = dinv * (s + hs_ref[...]) + b_ref[...]


def _readout_kernel(h2_ref, wlc_ref, blc_ref, wf1_ref, bf1_ref,
                    wf2_ref, bf2_ref, wf3_ref, bf3_ref, o_ref, acc_ref):
    k = pl.program_id(0)

    @pl.when(k == 0)
    def _():
        acc_ref[...] = jnp.zeros_like(acc_ref)

    acc_ref[...] += jnp.dot(h2_ref[...], wlc_ref[...],
                            preferred_element_type=jnp.float32)

    @pl.when(k == pl.num_programs(0) - 1)
    def _():
        g = acc_ref[...] + blc_ref[...]
        g = _leaky(jnp.dot(g, wf1_ref[...],
                           preferred_element_type=jnp.float32) + bf1_ref[...])
        g = _leaky(jnp.dot(g, wf2_ref[...],
                           preferred_element_type=jnp.float32) + bf2_ref[...])
        o_ref[...] = jnp.dot(g, wf3_ref[...],
                             preferred_element_type=jnp.float32) + bf3_ref[...]


def _mm_scale(xp, W1, deg3):
    grid = NP // BN
    return pl.pallas_call(
        _mm_scale_kernel,
        grid=(grid,),
        in_specs=[
            pl.BlockSpec((BN, H), lambda i: (i, 0)),
            pl.BlockSpec((H, H), lambda i: (0, 0)),
            pl.BlockSpec((NC, BN, 1), lambda i: (0, i, 0)),
        ],
        out_specs=pl.BlockSpec((BN, H), lambda i: (i, 0)),
        out_shape=jax.ShapeDtypeStruct((NP, H), jnp.float32),
    )(xp, W1, deg3)


def _mid(S1, hs1, deg3, W2, b1):
    grid = NP // BN
    return pl.pallas_call(
        _mid_kernel,
        grid=(grid,),
        in_specs=[
            pl.BlockSpec((NC, BN, H), lambda i: (0, i, 0)),
            pl.BlockSpec((BN, H), lambda i: (i, 0)),
            pl.BlockSpec((NC, BN, 1), lambda i: (0, i, 0)),
            pl.BlockSpec((H, H), lambda i: (0, 0)),
            pl.BlockSpec((1, H), lambda i: (0, 0)),
        ],
        out_specs=pl.BlockSpec((BN, H), lambda i: (i, 0)),
        out_shape=jax.ShapeDtypeStruct((NP, H), jnp.float32),
    )(S1, hs1, deg3, W2, b1)


def _fin(S2, hs2, deg3, b2):
    grid = NP // BN
    return pl.pallas_call(
        _fin_kernel,
        grid=(grid,),
        in_specs=[
            pl.BlockSpec((NC, BN, H), lambda i: (0, i, 0)),
            pl.BlockSpec((BN, H), lambda i: (i, 0)),
            pl.BlockSpec((NC, BN, 1), lambda i: (0, i, 0)),
            pl.BlockSpec((1, H), lambda i: (0, 0)),
        ],
        out_specs=pl.BlockSpec((BN, H), lambda i: (i, 0)),
        out_shape=jax.ShapeDtypeStruct((NP, H), jnp.float32),
    )(S2, hs2, deg3, b2)


def _readout(h2r, Wlc, blc, Wf1, bf1, Wf2, bf2, Wf3, bf3):
    BK = 2048
    grid = (NODE_SZ * H) // BK
    return pl.pallas_call(
        _readout_kernel,
        grid=(grid,),
        in_specs=[
            pl.BlockSpec((NG, BK), lambda k: (0, k)),
            pl.BlockSpec((BK, H), lambda k: (k, 0)),
            pl.BlockSpec((1, H), lambda k: (0, 0)),
            pl.BlockSpec((H, H // 2), lambda k: (0, 0)),
            pl.BlockSpec((1, H // 2), lambda k: (0, 0)),
            pl.BlockSpec((H // 2, H // 4), lambda k: (0, 0)),
            pl.BlockSpec((1, H // 4), lambda k: (0, 0)),
            pl.BlockSpec((H // 4, 1), lambda k: (0, 0)),
            pl.BlockSpec((1, 1), lambda k: (0, 0)),
        ],
        out_specs=pl.BlockSpec((NG, 1), lambda k: (0, 0)),
        out_shape=jax.ShapeDtypeStruct((NG, 1), jnp.float32),
        scratch_shapes=[pltpu.VMEM((NG, H), jnp.float32)],
    )(h2r, Wlc, blc, Wf1, bf1, Wf2, bf2, Wf3, bf3)


def kernel(x, edge_index, edge_weight, batch,
           W1, b1, W2, b2, Wlc, blc, Wf1, bf1, Wf2, bf2, Wf3, bf3):
    row, col = edge_index[0], edge_index[1]

    pad_e = EP - E
    pad_idx = (jnp.arange(pad_e, dtype=jnp.int32) * 2503 + 17) % N
    rowp = jnp.concatenate([row, pad_idx])
    colp = jnp.concatenate([col, pad_idx])
    ewp = jnp.concatenate([edge_weight, jnp.zeros((pad_e,), jnp.float32)])
    xp = jnp.pad(x, ((0, NP - N), (0, 0)))

    deg2, ewc = _deg_call(colp, ewp)
    deg3 = deg2.reshape(NC, NP, 1)

    hs1 = _mm_scale(xp, W1, deg3)
    S1 = _spmm_call(hs1, rowp, colp, ewc)
    hs2 = _mid(S1, hs1, deg3, W2, b1.reshape(1, H))
    S2 = _spmm_call(hs2, rowp, colp, ewc)
    h2 = _fin(S2, hs2, deg3, b2.reshape(1, H))

    h2r = h2[:N].reshape(NG, NODE_SZ * H)
    return _readout(h2r, Wlc, blc.reshape(1, H),
                    Wf1, bf1.reshape(1, H // 2),
                    Wf2, bf2.reshape(1, H // 4),
                    Wf3, bf3.reshape(1, 1))


# speedup vs baseline: 1.1062x; 1.0491x over previous
import functools

import jax
import jax.numpy as jnp
from jax import lax
from jax.experimental import pallas as pl
from jax.experimental.pallas import tpu as pltpu
from jax.experimental.pallas import tpu_sc as plsc

N = 10000
E = 320000
H = 128
NG = 25
NODE_SZ = 400

NC = 2
NS = 16
NW = NC * NS

NP = 10240
ROWS_PER_TILE = NP // NS
CHUNK = 128
EW_PER = 10112
NCHUNK = EW_PER // CHUNK
EP = EW_PER * NW
ACC_PER_TILE = N // NS

BN = 1024


def _leaky(v):
    return jnp.where(v >= 0, v, 0.01 * v)


_MESH = functools.partial(
    plsc.VectorSubcoreMesh, core_axis_name="c", subcore_axis_name="s",
    num_cores=NC, num_subcores=NS)


def _deg_body(col_hbm, ew_hbm, deg_out, ewc_out,
              col_v0, col_v1, col_v2, ew_all, zb_v, deg_sh,
              c0, c1, c2, d0, d1, d2):
    cid = lax.axis_index("c")
    sid = lax.axis_index("s")
    wid = sid * NC + cid
    ebase = wid * EW_PER
    col_v = (col_v0, col_v1, col_v2)
    csem = (c0, c1, c2)
    dsem = (d0, d1, d2)

    z16 = jnp.zeros((16,), jnp.float32)
    for i in range(ROWS_PER_TILE // 16):
        zb_v[pl.ds(i * 16, 16)] = z16
    pltpu.sync_copy(zb_v, deg_sh.at[pl.ds(sid * ROWS_PER_TILE, ROWS_PER_TILE)])

    pltpu.sync_copy(ew_hbm.at[pl.ds(ebase, EW_PER)], ew_all)

    @plsc.parallel_loop(0, EW_PER // 16, unroll=4)
    def _(i):
        sl = pl.ds(i * 16, 16)
        ew_all[sl] = jnp.maximum(ew_all[sl], 0.0)

    pltpu.sync_copy(ew_all, ewc_out.at[pl.ds(ebase, EW_PER)])
    plsc.subcore_barrier()

    def col_load(k, b):
        pltpu.async_copy(
            col_hbm.at[pl.ds(ebase + k * CHUNK, CHUNK)], col_v[b], csem[b])

    def col_wait(k, b):
        pltpu.make_async_copy(
            col_hbm.at[pl.ds(ebase + k * CHUNK, CHUNK)],
            col_v[b], csem[b]).wait()

    def scat(k, b):
        pltpu.async_copy(ew_all.at[pl.ds(k * CHUNK, CHUNK)],
                         deg_sh.at[col_v[b]], dsem[b], add=True)

    def scat_wait(k, b):
        pltpu.make_async_copy(ew_all.at[pl.ds(k * CHUNK, CHUNK)],
                              deg_sh.at[col_v[b]], dsem[b]).wait()

    col_load(0, 0)
    col_load(1, 1)
    col_wait(0, 0)
    scat(0, 0)
    col_load(2, 2)
    col_wait(1, 1)
    scat(1, 1)
    scat_wait(0, 0)
    col_load(3, 0)

    def steady(kk, carry):
        for j in range(3):
            k = 2 + kk * 3 + j
            b = (2 + j) % 3
            b2 = (1 + j) % 3
            col_wait(k, b)
            scat(k, b)
            scat_wait(k - 1, b2)
            col_load(k + 2, b2)
        return carry

    lax.fori_loop(0, (NCHUNK - 4) // 3, steady, 0)
    for k in (NCHUNK - 2, NCHUNK - 1):
        b = k % 3
        col_wait(k, b)
        scat(k, b)
    for j, b in ((NCHUNK - 3, (NCHUNK - 3) % 3), (NCHUNK - 2, (NCHUNK - 2) % 3),
                 (NCHUNK - 1, (NCHUNK - 1) % 3)):
        scat_wait(j, b)

    plsc.subcore_barrier()
    sl = pl.ds(sid * ROWS_PER_TILE, ROWS_PER_TILE)
    pltpu.sync_copy(deg_sh.at[sl], deg_out.at[cid, sl])


def _deg_call(colp, ewp):
    k = pl.kernel(
        _deg_body,
        out_type=(jax.ShapeDtypeStruct((NC, NP), jnp.float32),
                  jax.ShapeDtypeStruct((EP,), jnp.float32)),
        mesh=_MESH(),
        scratch_types=(
            [pltpu.VMEM((CHUNK,), jnp.int32) for _ in range(3)]
            + [pltpu.VMEM((EW_PER,), jnp.float32),
               pltpu.VMEM((ROWS_PER_TILE,), jnp.float32),
               pltpu.VMEM_SHARED((NP,), jnp.float32)]
            + [pltpu.SemaphoreType.DMA for _ in range(6)]
        ),
    )
    return k(colp, ewp)


NBUF = 3


def _make_spmm_body():
    def body(hs_hbm, row_hbm, col_hbm, w_hbm, out_hbm,
             rv0, rv1, rv2, cv0, cv1, cv2, wv0, wv1, wv2,
             rb0, rb1, rb2, acc_sh,
             i0, i1, i2, g0, g1, g2, s0, s1, s2):
        cid = lax.axis_index("c")
        sid = lax.axis_index("s")
        wid = sid * NC + cid
        ebase = wid * EW_PER
        row_v = (rv0, rv1, rv2)
        col_v = (cv0, cv1, cv2)
        w_v = (wv0, wv1, wv2)
        rows = (rb0, rb1, rb2)
        isem = (i0, i1, i2)
        gsem = (g0, g1, g2)
        ssem = (s0, s1, s2)

        z16 = jnp.zeros((16,), jnp.float32)

        def zrow(i, carry):
            for j in range(H // 16):
                rb0[i, pl.ds(j * 16, 16)] = z16
            return carry

        lax.fori_loop(0, CHUNK, zrow, 0)

        @pl.when(sid < NS - 1)
        def _():
            for kk in range(5):
                pltpu.sync_copy(
                    rb0, acc_sh.at[pl.ds(sid * 640 + kk * CHUNK, CHUNK)])

        @pl.when(sid == NS - 1)
        def _():
            for kk in range(3):
                pltpu.sync_copy(
                    rb0, acc_sh.at[pl.ds(9600 + kk * CHUNK, CHUNK)])
            pltpu.sync_copy(rb0.at[pl.ds(0, 16)], acc_sh.at[pl.ds(9984, 16)])

        plsc.subcore_barrier()

        def issue_idx(k, b):
            base = ebase + k * CHUNK
            pltpu.async_copy(row_hbm.at[pl.ds(base, CHUNK)], row_v[b], isem[b])
            pltpu.async_copy(col_hbm.at[pl.ds(base, CHUNK)], col_v[b], isem[b])
            pltpu.async_copy(w_hbm.at[pl.ds(base, CHUNK)], w_v[b], isem[b])

        def wait_idx(k, b):
            base = ebase + k * CHUNK
            pltpu.make_async_copy(row_hbm.at[pl.ds(base, CHUNK)],
                                  row_v[b], isem[b]).wait()
            pltpu.make_async_copy(col_hbm.at[pl.ds(base, CHUNK)],
                                  col_v[b], isem[b]).wait()
            pltpu.make_async_copy(w_hbm.at[pl.ds(base, CHUNK)],
                                  w_v[b], isem[b]).wait()

        def gstart(b):
            pltpu.async_copy(hs_hbm.at[row_v[b]], rows[b], gsem[b])

        def gwait(b):
            pltpu.make_async_copy(hs_hbm.at[row_v[b]], rows[b],
                                  gsem[b]).wait()

        def wait_scatter(b):
            pltpu.make_async_copy(rows[b], acc_sh.at[col_v[b]],
                                  ssem[b]).wait()

        def process(k, b):
            gwait(b)
            rv = rows[b]
            wv = w_v[b]

            @plsc.parallel_loop(0, CHUNK // 16, unroll=2)
            def _(g):
                sv = wv[pl.ds(g * 16, 16)]
                for l in range(16):
                    bc = jnp.full((16,), sv[l], dtype=jnp.float32)
                    e = g * 16 + l
                    for j in range(H // 16):
                        sl = pl.ds(j * 16, 16)
                        rv[e, sl] = rv[e, sl] * bc

            pltpu.async_copy(rv, acc_sh.at[col_v[b]], ssem[b], add=True)

        issue_idx(0, 0)
        issue_idx(1, 1)
        wait_idx(0, 0)
        gstart(0)
        wait_idx(1, 1)
        gstart(1)
        process(0, 0)
        issue_idx(2, 2)
        wait_idx(2, 2)
        gstart(2)
        process(1, 1)
        wait_scatter(0)
        issue_idx(3, 0)

        def steady(kk, carry):
            for j in range(3):
                k = 2 + kk * 3 + j
                b = (2 + j) % 3
                bn = j % 3
                b2 = (1 + j) % 3
                wait_idx(k + 1, bn)
                gstart(bn)
                process(k, b)
                wait_scatter(b2)
                issue_idx(k + 2, b2)
            return carry

        lax.fori_loop(0, (NCHUNK - 4) // 3, steady, 0)
        wait_idx(NCHUNK - 1, (NCHUNK - 1) % 3)
        gstart((NCHUNK - 1) % 3)
        process(NCHUNK - 2, (NCHUNK - 2) % 3)
        process(NCHUNK - 1, (NCHUNK - 1) % 3)
        for b in range(NBUF):
            wait_scatter(b)

        plsc.subcore_barrier()

        @pl.when(sid < NS - 1)
        def _():
            for kk in range(5):
                sl = pl.ds(sid * 640 + kk * CHUNK, CHUNK)
                pltpu.sync_copy(acc_sh.at[sl], out_hbm.at[cid, sl])

        @pl.when(sid == NS - 1)
        def _():
            for kk in range(3):
                sl = pl.ds(9600 + kk * CHUNK, CHUNK)
                pltpu.sync_copy(acc_sh.at[sl], out_hbm.at[cid, sl])
            sl = pl.ds(9984, 16)
            pltpu.sync_copy(acc_sh.at[sl], out_hbm.at[cid, sl])

    return body


def _spmm_call(hs, rowp, colp, ewc):
    k = pl.kernel(
        _make_spmm_body(),
        out_type=jax.ShapeDtypeStruct((NC, NP, H), jnp.float32),
        mesh=_MESH(),
        scratch_types=(
            [pltpu.VMEM((CHUNK,), jnp.int32) for _ in range(NBUF)]
            + [pltpu.VMEM((CHUNK,), jnp.int32) for _ in range(NBUF)]
            + [pltpu.VMEM((CHUNK,), jnp.float32) for _ in range(NBUF)]
            + [pltpu.VMEM((CHUNK, H), jnp.float32) for _ in range(NBUF)]
            + [pltpu.VMEM_SHARED((N, H), jnp.float32)]
            + [pltpu.SemaphoreType.DMA for _ in range(3 * NBUF)]
        ),
    )
    return k(hs, rowp, colp, ewc)


def _mm_scale_kernel(x_ref, w_ref, deg_ref, o_ref):
    h = jnp.dot(x_ref[...], w_ref[...], preferred_element_type=jnp.float32)
    dinv = lax.rsqrt(deg_ref[0] + deg_ref[1] + 1.0)
    o_ref[...] = h * dinv


def _mid_kernel(s_ref, hs_ref, deg_ref, w_ref, b_ref, o_ref):
    dinv = lax.rsqrt(deg_ref[0] + deg_ref[1] + 1.0)
    s = s_ref[0] + s_ref[1]
    h1 = _leaky(dinv * (s + hs_ref[...]) + b_ref[...])
    o_ref[...] = dinv * jnp.dot(h1, w_ref[...],
                                preferred_element_type=jnp.float32)


def _fin_kernel(s_ref, hs_ref, deg_ref, b_ref, o_ref):
    dinv = lax.rsqrt(deg_ref[0] + deg_ref[1] + 1.0)
    s = s_ref[0] + s_ref[1]
    o_ref[...] = dinv * (s + hs_ref[...]) + b_ref[...]


def _readout_kernel(h2_ref, wlc_ref, blc_ref, wf1_ref, bf1_ref,
                    wf2_ref, bf2_ref, wf3_ref, bf3_ref, o_ref, acc_ref):
    k = pl.program_id(0)

    @pl.when(k == 0)
    def _():
        acc_ref[...] = jnp.zeros_like(acc_ref)

    acc_ref[...] += jnp.dot(h2_ref[...], wlc_ref[...],
                            preferred_element_type=jnp.float32)

    @pl.when(k == pl.num_programs(0) - 1)
    def _():
        g = acc_ref[...] + blc_ref[...]
        g = _leaky(jnp.dot(g, wf1_ref[...],
                           preferred_element_type=jnp.float32) + bf1_ref[...])
        g = _leaky(jnp.dot(g, wf2_ref[...],
                           preferred_element_type=jnp.float32) + bf2_ref[...])
        o_ref[...] = jnp.dot(g, wf3_ref[...],
                             preferred_element_type=jnp.float32) + bf3_ref[...]


def _mm_scale(xp, W1, deg3):
    grid = NP // BN
    return pl.pallas_call(
        _mm_scale_kernel,
        grid=(grid,),
        in_specs=[
            pl.BlockSpec((BN, H), lambda i: (i, 0)),
            pl.BlockSpec((H, H), lambda i: (0, 0)),
            pl.BlockSpec((NC, BN, 1), lambda i: (0, i, 0)),
        ],
        out_specs=pl.BlockSpec((BN, H), lambda i: (i, 0)),
        out_shape=jax.ShapeDtypeStruct((NP, H), jnp.float32),
    )(xp, W1, deg3)


def _mid(S1, hs1, deg3, W2, b1):
    grid = NP // BN
    return pl.pallas_call(
        _mid_kernel,
        grid=(grid,),
        in_specs=[
            pl.BlockSpec((NC, BN, H), lambda i: (0, i, 0)),
            pl.BlockSpec((BN, H), lambda i: (i, 0)),
            pl.BlockSpec((NC, BN, 1), lambda i: (0, i, 0)),
            pl.BlockSpec((H, H), lambda i: (0, 0)),
            pl.BlockSpec((1, H), lambda i: (0, 0)),
        ],
        out_specs=pl.BlockSpec((BN, H), lambda i: (i, 0)),
        out_shape=jax.ShapeDtypeStruct((NP, H), jnp.float32),
    )(S1, hs1, deg3, W2, b1)


def _fin(S2, hs2, deg3, b2):
    grid = NP // BN
    return pl.pallas_call(
        _fin_kernel,
        grid=(grid,),
        in_specs=[
            pl.BlockSpec((NC, BN, H), lambda i: (0, i, 0)),
            pl.BlockSpec((BN, H), lambda i: (i, 0)),
            pl.BlockSpec((NC, BN, 1), lambda i: (0, i, 0)),
            pl.BlockSpec((1, H), lambda i: (0, 0)),
        ],
        out_specs=pl.BlockSpec((BN, H), lambda i: (i, 0)),
        out_shape=jax.ShapeDtypeStruct((NP, H), jnp.float32),
    )(S2, hs2, deg3, b2)


def _readout(h2r, Wlc, blc, Wf1, bf1, Wf2, bf2, Wf3, bf3):
    BK = 2048
    grid = (NODE_SZ * H) // BK
    return pl.pallas_call(
        _readout_kernel,
        grid=(grid,),
        in_specs=[
            pl.BlockSpec((NG, BK), lambda k: (0, k)),
            pl.BlockSpec((BK, H), lambda k: (k, 0)),
            pl.BlockSpec((1, H), lambda k: (0, 0)),
            pl.BlockSpec((H, H // 2), lambda k: (0, 0)),
            pl.BlockSpec((1, H // 2), lambda k: (0, 0)),
            pl.BlockSpec((H // 2, H // 4), lambda k: (0, 0)),
            pl.BlockSpec((1, H // 4), lambda k: (0, 0)),
            pl.BlockSpec((H // 4, 1), lambda k: (0, 0)),
            pl.BlockSpec((1, 1), lambda k: (0, 0)),
        ],
        out_specs=pl.BlockSpec((NG, 1), lambda k: (0, 0)),
        out_shape=jax.ShapeDtypeStruct((NG, 1), jnp.float32),
        scratch_shapes=[pltpu.VMEM((NG, H), jnp.float32)],
    )(h2r, Wlc, blc, Wf1, bf1, Wf2, bf2, Wf3, bf3)


def kernel(x, edge_index, edge_weight, batch,
           W1, b1, W2, b2, Wlc, blc, Wf1, bf1, Wf2, bf2, Wf3, bf3):
    row, col = edge_index[0], edge_index[1]

    pad_e = EP - E
    pad_idx = (jnp.arange(pad_e, dtype=jnp.int32) * 2503 + 17) % N
    rowp = jnp.concatenate([row, pad_idx])
    colp = jnp.concatenate([col, pad_idx])
    ewp = jnp.concatenate([edge_weight, jnp.zeros((pad_e,), jnp.float32)])
    xp = jnp.pad(x, ((0, NP - N), (0, 0)))

    deg2, ewc = _deg_call(colp, ewp)
    deg3 = deg2.reshape(NC, NP, 1)

    hs1 = _mm_scale(xp, W1, deg3)
    S1 = _spmm_call(hs1, rowp, colp, ewc)
    hs2 = _mid(S1, hs1, deg3, W2, b1.reshape(1, H))
    S2 = _spmm_call(hs2, rowp, colp, ewc)
    h2 = _fin(S2, hs2, deg3, b2.reshape(1, H))

    h2r = h2[:N].reshape(NG, NODE_SZ * H)
    return _readout(h2r, Wlc, blc.reshape(1, H),
                    Wf1, bf1.reshape(1, H // 2),
                    Wf2, bf2.reshape(1, H // 4),
                    Wf3, bf3.reshape(1, 1))
